# baseline (speedup 1.0000x reference)
import functools

import jax
import jax.numpy as jnp
from jax import lax
from jax.experimental import pallas as pl
from jax.experimental.pallas import tpu as pltpu
from jax.experimental.pallas import tpu_sc as plsc

VOCAB = 1000000
EMBED_DIM = 64
BATCH = 4096
SEQ = 200

_info = plsc.get_sparse_core_info()
_NC, _NS, _L = _info.num_cores, _info.num_subcores, _info.num_lanes
_NW = _NC * _NS


def _sc_body(x_hbm, pos_hbm, tok_hbm, out_hbm, idx_v, rows_v, pos_v, sem):
    wid = lax.axis_index("s") * _NC + lax.axis_index("c")
    rows_per_w = BATCH // _NW

    pltpu.sync_copy(pos_hbm, pos_v)

    def chunk_body(i, carry):
        base = (wid * rows_per_w + i) * SEQ
        pltpu.sync_copy(x_hbm.at[pl.ds(base, SEQ)], idx_v)
        pltpu.async_copy(tok_hbm.at[idx_v], rows_v, sem).wait()

        def add_body(r, c2):
            for j in range(EMBED_DIM // _L):
                sl = pl.ds(j * _L, _L)
                plsc.addupdate(rows_v.at[r, sl], pos_v[r, sl])
            return c2

        lax.fori_loop(0, SEQ, add_body, 0, unroll=2)
        pltpu.sync_copy(rows_v, out_hbm.at[pl.ds(base, SEQ)])
        return carry

    lax.fori_loop(0, rows_per_w, chunk_body, 0)


@jax.jit
def _embed_sc(x_flat, token_table, pos_table):
    mesh = plsc.VectorSubcoreMesh(core_axis_name="c", subcore_axis_name="s")
    k = pl.kernel(
        _sc_body,
        out_type=jax.ShapeDtypeStruct((BATCH * SEQ, EMBED_DIM), jnp.float32),
        mesh=mesh,
        scratch_types=[
            pltpu.VMEM((SEQ,), jnp.int32),
            pltpu.VMEM((SEQ, EMBED_DIM), jnp.float32),
            pltpu.VMEM((SEQ, EMBED_DIM), jnp.float32),
            pltpu.SemaphoreType.DMA,
        ],
        compiler_params=pltpu.CompilerParams(use_tc_tiling_on_sc=False),
    )
    return k(x_flat, pos_table, token_table)


def kernel(x, token_table, pos_table):
    x_flat = x.reshape(-1).astype(jnp.int32)
    out = _embed_sc(x_flat, token_table, pos_table)
    return out.reshape(BATCH, SEQ, EMBED_DIM)


# speedup vs baseline: 1.2075x; 1.2075x over previous
import functools

import jax
import jax.numpy as jnp
from jax import lax
from jax.experimental import pallas as pl
from jax.experimental.pallas import tpu as pltpu
from jax.experimental.pallas import tpu_sc as plsc

VOCAB = 1000000
EMBED_DIM = 64
BATCH = 4096
SEQ = 200

_info = plsc.get_sparse_core_info()
_NC, _NS, _L = _info.num_cores, _info.num_subcores, _info.num_lanes
_NW = _NC * _NS
_ROWS_PER_W = BATCH // _NW
_NBUF = 4


def _sc_body(x_hbm, pos_hbm, tok_hbm, out_hbm,
             idx_all, pos_v, rows, isem, gsems, ssems):
    wid = lax.axis_index("s") * _NC + lax.axis_index("c")
    row0 = wid * _ROWS_PER_W

    idx_cp = pltpu.async_copy(
        x_hbm.at[pl.ds(row0 * SEQ, _ROWS_PER_W * SEQ)], idx_all, isem)
    pltpu.sync_copy(pos_hbm, pos_v)
    idx_cp.wait()

    def start_gather(chunk, b):
        pltpu.async_copy(
            tok_hbm.at[idx_all.at[pl.ds(chunk * SEQ, SEQ)]], rows[b],
            gsems[b])

    def wait_gather(chunk, b):
        pltpu.make_async_copy(
            tok_hbm.at[idx_all.at[pl.ds(chunk * SEQ, SEQ)]], rows[b],
            gsems[b]).wait()

    def start_store(chunk, b):
        pltpu.async_copy(
            rows[b], out_hbm.at[pl.ds((row0 + chunk) * SEQ, SEQ)], ssems[b])

    def wait_store(chunk, b):
        pltpu.make_async_copy(
            rows[b], out_hbm.at[pl.ds((row0 + chunk) * SEQ, SEQ)],
            ssems[b]).wait()

    start_gather(0, 0)
    start_gather(1, 1)

    def outer(o, carry):
        for b in range(_NBUF):
            i = o * _NBUF + b
            nxt = i + 2
            bn = (b + 2) % _NBUF

            @pl.when(nxt < _ROWS_PER_W)
            def _():
                @pl.when(i >= 2)
                def _():
                    wait_store(i - 2, bn)
                start_gather(nxt, bn)

            wait_gather(i, b)

            rbuf = rows[b]

            @plsc.parallel_loop(0, SEQ, unroll=4)
            def _(r):
                for j in range(EMBED_DIM // _L):
                    sl = pl.ds(j * _L, _L)
                    plsc.addupdate(rbuf.at[r, sl], pos_v[r, sl])

            start_store(i, b)
        return carry

    lax.fori_loop(0, _ROWS_PER_W // _NBUF, outer, 0)

    for b in range(_NBUF):
        i = _ROWS_PER_W - _NBUF + b
        wait_store(i, b)


@jax.jit
def _embed_sc(x_flat, token_table, pos_table):
    mesh = plsc.VectorSubcoreMesh(core_axis_name="c", subcore_axis_name="s")
    k = pl.kernel(
        _sc_body,
        out_type=jax.ShapeDtypeStruct((BATCH * SEQ, EMBED_DIM), jnp.float32),
        mesh=mesh,
        scratch_types=[
            pltpu.VMEM((_ROWS_PER_W * SEQ,), jnp.int32),
            pltpu.VMEM((SEQ, EMBED_DIM), jnp.float32),
            [pltpu.VMEM((SEQ, EMBED_DIM), jnp.float32)] * _NBUF,
            pltpu.SemaphoreType.DMA,
            [pltpu.SemaphoreType.DMA] * _NBUF,
            [pltpu.SemaphoreType.DMA] * _NBUF,
        ],
        compiler_params=pltpu.CompilerParams(use_tc_tiling_on_sc=False),
    )
    return k(x_flat, pos_table, token_table)


def kernel(x, token_table, pos_table):
    x_flat = x.reshape(-1).astype(jnp.int32)
    out = _embed_sc(x_flat, token_table, pos_table)
    return out.reshape(BATCH, SEQ, EMBED_DIM)
